# baseline (speedup 1.0000x reference)
import jax
import jax.numpy as jnp
from jax import lax
from jax.experimental import pallas as pl
from jax.experimental.pallas import tpu as pltpu
from jax.experimental.pallas import tpu_sc as plsc

_MU = 3.5
_B = 16384
_K = 90
_D = 128
_NC = 2
_NS = 16
_NW = _NC * _NS
_BPW = _B // _NW
_CH = 128
_NCHUNK = _BPW // _CH
_L = 16
_NG = _BPW // _L
_DEPTH = 4


def _sc_body(p_hbm, q_hbm, bu_hbm, bi_hbm, uidx_hbm, iidx_hbm, out_hbm,
             uidx_v, iidx_v, p_v, q_v, bu_v, bi_v, out_v, bsem,
             sem_a, sem_b, sem_c, sem_d):
    c = lax.axis_index("c")
    s = lax.axis_index("s")
    wid = s * _NC + c

    pltpu.sync_copy(uidx_hbm.at[pl.ds(wid * _NCHUNK, _NCHUNK)], uidx_v)
    pltpu.sync_copy(iidx_hbm.at[pl.ds(wid * _NCHUNK, _NCHUNK)], iidx_v)

    bias_copies = []
    for j in range(_NCHUNK):
        dst = pl.ds(j * _CH, _CH)
        bias_copies.append(pltpu.async_copy(bu_hbm.at[uidx_v.at[j]], bu_v.at[dst], bsem))
        bias_copies.append(pltpu.async_copy(bi_hbm.at[iidx_v.at[j]], bi_v.at[dst], bsem))

    lane = lax.iota(jnp.int32, _L)
    tail_mask = lane >= 6
    zero = jnp.zeros((_L,), jnp.float32)

    def issue(g, slot, sem):
        j0 = g // 8
        off = (g % 8) * _L
        uvec = uidx_v[j0, pl.ds(off, _L)]
        ivec = iidx_v[j0, pl.ds(off, _L)]
        for j in range(_L):
            u = uvec[j]
            i = ivec[j]
            row = slot * _L + j
            pltpu.async_copy(p_hbm.at[pl.ds(u, 1), :],
                             p_v.at[pl.ds(row, 1), pl.ds(0, _K)], sem)
            pltpu.async_copy(q_hbm.at[pl.ds(i, 1), :],
                             q_v.at[pl.ds(row, 1), pl.ds(0, _K)], sem)

    def drain(slot, sem):
        base = slot * _L
        pltpu.make_async_copy(p_hbm.at[pl.ds(0, _L), :],
                              p_v.at[pl.ds(base, _L), pl.ds(0, _K)], sem).wait()
        pltpu.make_async_copy(q_hbm.at[pl.ds(0, _L), :],
                              q_v.at[pl.ds(base, _L), pl.ds(0, _K)], sem).wait()

    def compute(g, slot):
        res = zero
        for j in range(_L):
            r = slot * _L + j
            acc = p_v[r, pl.ds(0, _L)] * q_v[r, pl.ds(0, _L)]
            for cix in range(1, 5):
                acc = acc + p_v[r, pl.ds(cix * _L, _L)] * q_v[r, pl.ds(cix * _L, _L)]
            tp = p_v[r, pl.ds(74, _L)] * q_v[r, pl.ds(74, _L)]
            acc = acc + jnp.where(tail_mask, tp, zero)
            res = jnp.where(lane == j, jnp.sum(acc), res)
        base = g * _L
        out_v[pl.ds(base, _L)] = (res + bu_v[pl.ds(base, _L)]
                                  + bi_v[pl.ds(base, _L)] + jnp.float32(_MU))
        return res

    for cp in bias_copies:
        cp.wait()

    sems = [sem_a, sem_b, sem_c, sem_d]
    for k in range(_DEPTH - 1):
        issue(jnp.int32(k), k, sems[k])

    def quad(g4, carry):
        for k in range(_DEPTH):
            g = g4 * _DEPTH + k
            ahead = g + _DEPTH - 1

            @pl.when(ahead < _NG)
            def _(ahead=ahead, k=k):
                issue(ahead, (k + _DEPTH - 1) % _DEPTH, sems[(k + _DEPTH - 1) % _DEPTH])

            drain(k, sems[k])
            compute(g, k)
        return carry

    lax.fori_loop(0, _NG // _DEPTH, quad, 0)

    pltpu.sync_copy(out_v, out_hbm.at[pl.ds(wid * _BPW, _BPW)])


@jax.jit
def _run(P, Q, b_u, b_i, uidx2, iidx2):
    mesh = plsc.VectorSubcoreMesh(core_axis_name="c", subcore_axis_name="s")
    f = pl.kernel(
        _sc_body,
        out_type=jax.ShapeDtypeStruct((_B,), jnp.float32),
        mesh=mesh,
        compiler_params=pltpu.CompilerParams(needs_layout_passes=False),
        scratch_types=[
            pltpu.VMEM((_NCHUNK, _CH), jnp.int32),
            pltpu.VMEM((_NCHUNK, _CH), jnp.int32),
            pltpu.VMEM((_DEPTH * _L, _K), jnp.float32),
            pltpu.VMEM((_DEPTH * _L, _K), jnp.float32),
            pltpu.VMEM((_BPW,), jnp.float32),
            pltpu.VMEM((_BPW,), jnp.float32),
            pltpu.VMEM((_BPW,), jnp.float32),
            pltpu.SemaphoreType.DMA,
            pltpu.SemaphoreType.DMA,
            pltpu.SemaphoreType.DMA,
            pltpu.SemaphoreType.DMA,
            pltpu.SemaphoreType.DMA,
        ],
    )
    return f(P, Q, b_u, b_i, uidx2, iidx2)


def kernel(P, Q, b_u, b_i, user_idx, item_idx):
    uidx2 = user_idx.astype(jnp.int32).reshape(_B // _CH, _CH)
    iidx2 = item_idx.astype(jnp.int32).reshape(_B // _CH, _CH)
    return _run(P, Q, b_u.reshape(-1), b_i.reshape(-1), uidx2, iidx2)


# speedup vs baseline: 1.0544x; 1.0544x over previous
import jax
import jax.numpy as jnp
from jax import lax
from jax.experimental import pallas as pl
from jax.experimental.pallas import tpu as pltpu
from jax.experimental.pallas import tpu_sc as plsc

_MU = 3.5
_B = 16384
_K = 90
_D = 128
_NC = 2
_NS = 16
_NW = _NC * _NS
_BPW = _B // _NW
_CH = 128
_NCHUNK = _BPW // _CH
_L = 16
_NG = _BPW // _L
_DEPTH = 2


def _sc_body(p_hbm, q_hbm, bu_hbm, bi_hbm, uidx_hbm, iidx_hbm, out_hbm,
             uidx_v, iidx_v, p_v, q_v, bu_v, bi_v, out_v, bsem,
             sem_a, sem_b, sem_c, sem_d):
    c = lax.axis_index("c")
    s = lax.axis_index("s")
    wid = s * _NC + c

    pltpu.sync_copy(uidx_hbm.at[pl.ds(wid * _NCHUNK, _NCHUNK)], uidx_v)
    pltpu.sync_copy(iidx_hbm.at[pl.ds(wid * _NCHUNK, _NCHUNK)], iidx_v)

    bias_copies = []
    for j in range(_NCHUNK):
        dst = pl.ds(j * _CH, _CH)
        bias_copies.append(pltpu.async_copy(bu_hbm.at[uidx_v.at[j]], bu_v.at[dst], bsem))
        bias_copies.append(pltpu.async_copy(bi_hbm.at[iidx_v.at[j]], bi_v.at[dst], bsem))

    lane = lax.iota(jnp.int32, _L)
    tail_mask = lane >= 6
    zero = jnp.zeros((_L,), jnp.float32)

    def issue(g, slot, sem):
        j0 = g // 8
        off = (g % 8) * _L
        uvec = uidx_v[j0, pl.ds(off, _L)]
        ivec = iidx_v[j0, pl.ds(off, _L)]
        for j in range(_L):
            u = uvec[j]
            i = ivec[j]
            row = slot * _L + j
            pltpu.async_copy(p_hbm.at[pl.ds(u, 1), :],
                             p_v.at[pl.ds(row, 1), pl.ds(0, _K)], sem)
            pltpu.async_copy(q_hbm.at[pl.ds(i, 1), :],
                             q_v.at[pl.ds(row, 1), pl.ds(0, _K)], sem)

    def drain(slot, sem):
        base = slot * _L
        pltpu.make_async_copy(p_hbm.at[pl.ds(0, _L), :],
                              p_v.at[pl.ds(base, _L), pl.ds(0, _K)], sem).wait()
        pltpu.make_async_copy(q_hbm.at[pl.ds(0, _L), :],
                              q_v.at[pl.ds(base, _L), pl.ds(0, _K)], sem).wait()

    def compute(g, slot):
        res = zero
        for j in range(_L):
            r = slot * _L + j
            acc = p_v[r, pl.ds(0, _L)] * q_v[r, pl.ds(0, _L)]
            for cix in range(1, 5):
                acc = acc + p_v[r, pl.ds(cix * _L, _L)] * q_v[r, pl.ds(cix * _L, _L)]
            tp = p_v[r, pl.ds(74, _L)] * q_v[r, pl.ds(74, _L)]
            acc = acc + jnp.where(tail_mask, tp, zero)
            res = jnp.where(lane == j, jnp.sum(acc), res)
        base = g * _L
        out_v[pl.ds(base, _L)] = (res + bu_v[pl.ds(base, _L)]
                                  + bi_v[pl.ds(base, _L)] + jnp.float32(_MU))
        return res

    for cp in bias_copies:
        cp.wait()

    sems = [sem_a, sem_b, sem_c, sem_d]
    for k in range(_DEPTH - 1):
        issue(jnp.int32(k), k, sems[k])

    def quad(g4, carry):
        for k in range(_DEPTH):
            g = g4 * _DEPTH + k
            ahead = g + _DEPTH - 1

            @pl.when(ahead < _NG)
            def _(ahead=ahead, k=k):
                issue(ahead, (k + _DEPTH - 1) % _DEPTH, sems[(k + _DEPTH - 1) % _DEPTH])

            drain(k, sems[k])
            compute(g, k)
        return carry

    lax.fori_loop(0, _NG // _DEPTH, quad, 0)

    pltpu.sync_copy(out_v, out_hbm.at[pl.ds(wid * _BPW, _BPW)])


@jax.jit
def _run(P, Q, b_u, b_i, uidx2, iidx2):
    mesh = plsc.VectorSubcoreMesh(core_axis_name="c", subcore_axis_name="s")
    f = pl.kernel(
        _sc_body,
        out_type=jax.ShapeDtypeStruct((_B,), jnp.float32),
        mesh=mesh,
        compiler_params=pltpu.CompilerParams(needs_layout_passes=False),
        scratch_types=[
            pltpu.VMEM((_NCHUNK, _CH), jnp.int32),
            pltpu.VMEM((_NCHUNK, _CH), jnp.int32),
            pltpu.VMEM((_DEPTH * _L, _K), jnp.float32),
            pltpu.VMEM((_DEPTH * _L, _K), jnp.float32),
            pltpu.VMEM((_BPW,), jnp.float32),
            pltpu.VMEM((_BPW,), jnp.float32),
            pltpu.VMEM((_BPW,), jnp.float32),
            pltpu.SemaphoreType.DMA,
            pltpu.SemaphoreType.DMA,
            pltpu.SemaphoreType.DMA,
            pltpu.SemaphoreType.DMA,
            pltpu.SemaphoreType.DMA,
        ],
    )
    return f(P, Q, b_u, b_i, uidx2, iidx2)


def kernel(P, Q, b_u, b_i, user_idx, item_idx):
    uidx2 = user_idx.astype(jnp.int32).reshape(_B // _CH, _CH)
    iidx2 = item_idx.astype(jnp.int32).reshape(_B // _CH, _CH)
    return _run(P, Q, b_u.reshape(-1), b_i.reshape(-1), uidx2, iidx2)
